# baseline (speedup 1.0000x reference)
import functools

import jax
import jax.numpy as jnp
from jax.experimental import pallas as pl
from jax.experimental.pallas import tpu as pltpu

TAU = 0.05


def _encoder_kernel(ht_ref, hc_ref, o_ref, sh_ref, w1_ref, b1_ref, w2_ref,
                    b2_ref, out_ref, *, nh, no, kqs):
    f32 = jnp.float32
    bf16 = jnp.bfloat16
    rp = lambda x: x.astype(bf16).astype(f32)
    h3t = ht_ref[0]
    hx = h3t[0:1, :]
    hy = h3t[1:2, :]
    hz = h3t[2:3, :]
    h3c = hc_ref[0]
    o3 = o_ref[0]

    a2 = hx * hx + hy * hy + hz * hz
    b2 = jnp.sum(o3 * o3, axis=1, keepdims=True)
    cross = jnp.dot(o3.astype(bf16), h3t.astype(bf16),
                    preferred_element_type=f32)
    sq = (a2 + b2) - 2.0 * cross

    min_sq_h = jnp.min(sq, axis=0, keepdims=True)
    dmin_h = jnp.sqrt(jnp.maximum(min_sq_h, 1e-12))
    min_sq_o = jnp.min(sq, axis=1, keepdims=True)
    dmin_o = jnp.sqrt(jnp.maximum(min_sq_o, 1e-12))

    iota_m = jax.lax.broadcasted_iota(jnp.int32, (no, nh), 0)
    first = jnp.min(jnp.where(sq == min_sq_h, iota_m, no),
                    axis=0, keepdims=True)
    packed = jnp.concatenate([min_sq_h, first.astype(f32)], axis=0)
    packed_t = jnp.transpose(packed)
    kcol = packed_t[:, 0:1]
    first_col = packed_t[:, 1:2]

    jj = jax.lax.broadcasted_iota(jnp.int32, (nh, no), 1)
    mask_t = (jj.astype(f32) == first_col).astype(f32)
    o_nn = jnp.dot(mask_t, o3, preferred_element_type=f32,
                   precision=jax.lax.Precision.HIGHEST)
    vec = o_nn - h3c
    nrm = jnp.sqrt(jnp.maximum(
        jnp.sum(vec * vec, axis=1, keepdims=True), 1e-6))
    dirsum = jnp.sum(vec / nrm, axis=0, keepdims=True)

    sh = sh_ref[0]
    w_h = jnp.exp(-dmin_h * (1.0 / TAU)) * sh

    ii = jax.lax.broadcasted_iota(jnp.int32, (nh, nh), 0)
    jj2 = jax.lax.broadcasted_iota(jnp.int32, (nh, nh), 1)
    lt = (kcol < min_sq_h).astype(f32)
    tie = ((kcol == min_sq_h) & (ii < jj2)).astype(f32)
    rank = jnp.sum(lt + tie, axis=0, keepdims=True)

    inv_nh = 1.0 / nh
    f1 = jnp.sum(dmin_h, keepdims=True) * inv_nh
    f2 = jnp.min(dmin_h, keepdims=True)
    q = []
    for kq in kqs:
        sel = (rank < float(kq)).astype(f32)
        q.append(jnp.sum(dmin_h * sel, keepdims=True) * (1.0 / kq))
    f6 = jnp.sum(w_h, keepdims=True) * inv_nh
    f7 = dirsum[0:1, 0:1] * inv_nh
    f8 = dirsum[0:1, 1:2] * inv_nh
    f9 = dirsum[0:1, 2:3] * inv_nh
    f10 = jnp.sum(dmin_o, keepdims=True) * (1.0 / no)

    feats = (f1, f2, q[0], q[1], q[2], f6, f7, f8, f9, f10)
    w1 = rp(w1_ref[:])
    acc = b1_ref[:]
    for k, f in enumerate(feats):
        acc = acc + rp(f) * w1[k:k + 1, :]
    hid = jnp.maximum(acc, 0.0)
    out = jnp.dot(hid.astype(bf16), w2_ref[:].astype(bf16),
                  preferred_element_type=f32) + b2_ref[:]
    out_ref[0] = out


def kernel(human_bt_n3, object_bt_m3, s_h_bt_n, s_o_bt_m, W1, b1, W2, b2):
    B, T, Nh, _ = human_bt_n3.shape
    No = object_bt_m3.shape[2]
    BT = B * T
    Dout = W2.shape[1]
    h = human_bt_n3.reshape(BT, Nh, 3)
    ht = h.transpose(0, 2, 1)
    o = object_bt_m3.reshape(BT, No, 3)
    sh = s_h_bt_n.reshape(BT, 1, Nh)
    b1r = b1.reshape(1, -1)
    b2r = b2.reshape(1, -1)
    kqs = tuple(int(max(1, round(qv * Nh))) for qv in (0.2, 0.5, 0.8))

    body = functools.partial(_encoder_kernel, nh=Nh, no=No, kqs=kqs)
    out = pl.pallas_call(
        body,
        grid=(BT,),
        in_specs=[
            pl.BlockSpec((1, 3, Nh), lambda i: (i, 0, 0)),
            pl.BlockSpec((1, Nh, 3), lambda i: (i, 0, 0)),
            pl.BlockSpec((1, No, 3), lambda i: (i, 0, 0)),
            pl.BlockSpec((1, 1, Nh), lambda i: (i, 0, 0)),
            pl.BlockSpec(W1.shape, lambda i: (0, 0)),
            pl.BlockSpec(b1r.shape, lambda i: (0, 0)),
            pl.BlockSpec(W2.shape, lambda i: (0, 0)),
            pl.BlockSpec(b2r.shape, lambda i: (0, 0)),
        ],
        out_specs=pl.BlockSpec((1, 1, Dout), lambda i: (i, 0, 0)),
        out_shape=jax.ShapeDtypeStruct((BT, 1, Dout), jnp.float32),
        compiler_params=pltpu.CompilerParams(
            dimension_semantics=("parallel",)),
    )(ht, h, o, sh, W1, b1r, W2, b2r)
    return out.reshape(B, T, Dout)


# speedup vs baseline: 1.0137x; 1.0137x over previous
import functools

import jax
import jax.numpy as jnp
from jax.experimental import pallas as pl
from jax.experimental.pallas import tpu as pltpu

TAU = 0.05


def _encoder_kernel(hc_ref, ot_ref, o_ref, sh_ref, w1_ref, b1_ref, w2_ref,
                    b2_ref, out_ref, *, nh, no, kqs):
    f32 = jnp.float32
    bf16 = jnp.bfloat16
    rp = lambda x: x.astype(bf16).astype(f32)
    h3c = hc_ref[0]
    o3t = ot_ref[0]
    o3 = o_ref[0]

    a2c = jnp.sum(h3c * h3c, axis=1, keepdims=True)
    oxr = o3t[0:1, :]
    oyr = o3t[1:2, :]
    ozr = o3t[2:3, :]
    b2r = oxr * oxr + oyr * oyr + ozr * ozr
    cross = jnp.dot(h3c.astype(bf16), o3t.astype(bf16),
                    preferred_element_type=f32)
    sq = (a2c + b2r) - 2.0 * cross

    kcol = jnp.min(sq, axis=1, keepdims=True)
    min_sq_o = jnp.min(sq, axis=0, keepdims=True)
    dmin_o = jnp.sqrt(jnp.maximum(min_sq_o, 1e-12))

    jj = jax.lax.broadcasted_iota(jnp.int32, (nh, no), 1)
    first_col = jnp.min(jnp.where(sq == kcol, jj, no),
                        axis=1, keepdims=True)
    mask = (jj == first_col).astype(bf16)
    c1 = o3.astype(bf16)
    r1 = o3 - c1.astype(f32)
    c2 = r1.astype(bf16)
    c3 = (r1 - c2.astype(f32)).astype(bf16)
    chunks = jnp.concatenate([c1, c2, c3], axis=1)
    g = jnp.dot(mask, chunks, preferred_element_type=f32)
    o_nn = g[:, 0:3] + g[:, 3:6] + g[:, 6:9]
    vec = o_nn - h3c
    nrm = jnp.sqrt(jnp.maximum(
        jnp.sum(vec * vec, axis=1, keepdims=True), 1e-6))
    dirsum = jnp.sum(vec / nrm, axis=0, keepdims=True)

    krow = jnp.transpose(kcol)
    dmin_h = jnp.sqrt(jnp.maximum(krow, 1e-12))
    sh = sh_ref[0]
    w_h = jnp.exp(-dmin_h * (1.0 / TAU)) * sh

    ii = jax.lax.broadcasted_iota(jnp.int32, (nh, nh), 0)
    jj2 = jax.lax.broadcasted_iota(jnp.int32, (nh, nh), 1)
    lt = (kcol < krow).astype(f32)
    tie = ((kcol == krow) & (ii < jj2)).astype(f32)
    rank = jnp.sum(lt + tie, axis=0, keepdims=True)

    inv_nh = 1.0 / nh
    f1 = jnp.sum(dmin_h, keepdims=True) * inv_nh
    f2 = jnp.min(dmin_h, keepdims=True)
    q = []
    for kq in kqs:
        sel = (rank < float(kq)).astype(f32)
        q.append(jnp.sum(dmin_h * sel, keepdims=True) * (1.0 / kq))
    f6 = jnp.sum(w_h, keepdims=True) * inv_nh
    f7 = dirsum[0:1, 0:1] * inv_nh
    f8 = dirsum[0:1, 1:2] * inv_nh
    f9 = dirsum[0:1, 2:3] * inv_nh
    f10 = jnp.sum(dmin_o, keepdims=True) * (1.0 / no)

    feats = (f1, f2, q[0], q[1], q[2], f6, f7, f8, f9, f10)
    w1 = rp(w1_ref[:])
    acc = b1_ref[:]
    for k, f in enumerate(feats):
        acc = acc + rp(f) * w1[k:k + 1, :]
    hid = jnp.maximum(acc, 0.0)
    out = jnp.dot(hid.astype(bf16), w2_ref[:].astype(bf16),
                  preferred_element_type=f32) + b2_ref[:]
    out_ref[0] = out


def kernel(human_bt_n3, object_bt_m3, s_h_bt_n, s_o_bt_m, W1, b1, W2, b2):
    B, T, Nh, _ = human_bt_n3.shape
    No = object_bt_m3.shape[2]
    BT = B * T
    Dout = W2.shape[1]
    h = human_bt_n3.reshape(BT, Nh, 3)
    o = object_bt_m3.reshape(BT, No, 3)
    ot = o.transpose(0, 2, 1)
    sh = s_h_bt_n.reshape(BT, 1, Nh)
    b1r = b1.reshape(1, -1)
    b2r = b2.reshape(1, -1)
    kqs = tuple(int(max(1, round(qv * Nh))) for qv in (0.2, 0.5, 0.8))

    body = functools.partial(_encoder_kernel, nh=Nh, no=No, kqs=kqs)
    out = pl.pallas_call(
        body,
        grid=(BT,),
        in_specs=[
            pl.BlockSpec((1, Nh, 3), lambda i: (i, 0, 0)),
            pl.BlockSpec((1, 3, No), lambda i: (i, 0, 0)),
            pl.BlockSpec((1, No, 3), lambda i: (i, 0, 0)),
            pl.BlockSpec((1, 1, Nh), lambda i: (i, 0, 0)),
            pl.BlockSpec(W1.shape, lambda i: (0, 0)),
            pl.BlockSpec(b1r.shape, lambda i: (0, 0)),
            pl.BlockSpec(W2.shape, lambda i: (0, 0)),
            pl.BlockSpec(b2r.shape, lambda i: (0, 0)),
        ],
        out_specs=pl.BlockSpec((1, 1, Dout), lambda i: (i, 0, 0)),
        out_shape=jax.ShapeDtypeStruct((BT, 1, Dout), jnp.float32),
        compiler_params=pltpu.CompilerParams(
            dimension_semantics=("parallel",)),
    )(h, ot, o, sh, W1, b1r, W2, b2r)
    return out.reshape(B, T, Dout)


# speedup vs baseline: 2.0067x; 1.9795x over previous
import functools

import jax
import jax.numpy as jnp
from jax.experimental import pallas as pl
from jax.experimental.pallas import tpu as pltpu

TAU = 0.05


def _encoder_kernel(ht_ref, o_ref, ot_ref, sh_ref, w1_ref, b1_ref, w2_ref,
                    b2_ref, out_ref, *, nh, no, kqs):
    f32 = jnp.float32
    bf16 = jnp.bfloat16
    rp = lambda x: x.astype(bf16).astype(f32)
    h3t = ht_ref[0]
    hx = h3t[0:1, :]
    hy = h3t[1:2, :]
    hz = h3t[2:3, :]
    o3 = o_ref[0]
    o3t = ot_ref[0]

    a2 = hx * hx + hy * hy + hz * hz
    b2c = jnp.sum(o3 * o3, axis=1, keepdims=True)
    cross = jnp.dot(o3.astype(bf16), h3t.astype(bf16),
                    preferred_element_type=f32)
    sq = (a2 + b2c) - 2.0 * cross

    min_sq_h = jnp.min(sq, axis=0, keepdims=True)
    dmin_h = jnp.sqrt(jnp.maximum(min_sq_h, 1e-12))
    min_sq_o = jnp.min(sq, axis=1, keepdims=True)
    dmin_o = jnp.sqrt(jnp.maximum(min_sq_o, 1e-12))

    ii = jax.lax.broadcasted_iota(jnp.int32, (no, nh), 0)
    first = jnp.min(jnp.where(sq == min_sq_h, ii, no),
                    axis=0, keepdims=True)
    mask = (ii == first).astype(bf16)
    c1 = o3t.astype(bf16)
    r1 = o3t - c1.astype(f32)
    c2 = r1.astype(bf16)
    c3 = (r1 - c2.astype(f32)).astype(bf16)
    chunks_t = jnp.concatenate([c1, c2, c3], axis=0)
    g = jnp.dot(chunks_t, mask, preferred_element_type=f32)
    onx = g[0:1, :] + g[3:4, :] + g[6:7, :]
    ony = g[1:2, :] + g[4:5, :] + g[7:8, :]
    onz = g[2:3, :] + g[5:6, :] + g[8:9, :]
    vx = onx - hx
    vy = ony - hy
    vz = onz - hz
    nrm = jnp.sqrt(jnp.maximum(vx * vx + vy * vy + vz * vz, 1e-6))

    sh = sh_ref[0]
    w_h = jnp.exp(-dmin_h * (1.0 / TAU)) * sh

    kcol = jnp.transpose(min_sq_h)
    i2 = jax.lax.broadcasted_iota(jnp.int32, (nh, nh), 0)
    j2 = jax.lax.broadcasted_iota(jnp.int32, (nh, nh), 1)
    cmp = ((kcol < min_sq_h)
           | ((kcol == min_sq_h) & (i2 < j2))).astype(bf16)
    rank = jnp.dot(jnp.ones((1, nh), bf16), cmp,
                   preferred_element_type=f32)

    inv_nh = 1.0 / nh
    f1 = jnp.sum(dmin_h, keepdims=True) * inv_nh
    f2 = jnp.min(dmin_h, keepdims=True)
    q = []
    for kq in kqs:
        sel = (rank < float(kq)).astype(f32)
        q.append(jnp.sum(dmin_h * sel, keepdims=True) * (1.0 / kq))
    f6 = jnp.sum(w_h, keepdims=True) * inv_nh
    f7 = jnp.sum(vx / nrm, keepdims=True) * inv_nh
    f8 = jnp.sum(vy / nrm, keepdims=True) * inv_nh
    f9 = jnp.sum(vz / nrm, keepdims=True) * inv_nh
    f10 = jnp.sum(dmin_o, keepdims=True) * (1.0 / no)

    feats = (f1, f2, q[0], q[1], q[2], f6, f7, f8, f9, f10)
    w1 = rp(w1_ref[:])
    acc = b1_ref[:]
    for k, f in enumerate(feats):
        acc = acc + rp(f) * w1[k:k + 1, :]
    hid = jnp.maximum(acc, 0.0)
    out = jnp.dot(hid.astype(bf16), w2_ref[:].astype(bf16),
                  preferred_element_type=f32) + b2_ref[:]
    out_ref[0] = out


def kernel(human_bt_n3, object_bt_m3, s_h_bt_n, s_o_bt_m, W1, b1, W2, b2):
    B, T, Nh, _ = human_bt_n3.shape
    No = object_bt_m3.shape[2]
    BT = B * T
    Dout = W2.shape[1]
    ht = human_bt_n3.reshape(BT, Nh, 3).transpose(0, 2, 1)
    o = object_bt_m3.reshape(BT, No, 3)
    ot = o.transpose(0, 2, 1)
    sh = s_h_bt_n.reshape(BT, 1, Nh)
    b1r = b1.reshape(1, -1)
    b2r = b2.reshape(1, -1)
    kqs = tuple(int(max(1, round(qv * Nh))) for qv in (0.2, 0.5, 0.8))

    body = functools.partial(_encoder_kernel, nh=Nh, no=No, kqs=kqs)
    out = pl.pallas_call(
        body,
        grid=(BT,),
        in_specs=[
            pl.BlockSpec((1, 3, Nh), lambda i: (i, 0, 0)),
            pl.BlockSpec((1, No, 3), lambda i: (i, 0, 0)),
            pl.BlockSpec((1, 3, No), lambda i: (i, 0, 0)),
            pl.BlockSpec((1, 1, Nh), lambda i: (i, 0, 0)),
            pl.BlockSpec(W1.shape, lambda i: (0, 0)),
            pl.BlockSpec(b1r.shape, lambda i: (0, 0)),
            pl.BlockSpec(W2.shape, lambda i: (0, 0)),
            pl.BlockSpec(b2r.shape, lambda i: (0, 0)),
        ],
        out_specs=pl.BlockSpec((1, 1, Dout), lambda i: (i, 0, 0)),
        out_shape=jax.ShapeDtypeStruct((BT, 1, Dout), jnp.float32),
        compiler_params=pltpu.CompilerParams(
            dimension_semantics=("parallel",)),
    )(ht, o, ot, sh, W1, b1r, W2, b2r)
    return out.reshape(B, T, Dout)
